# baseline (speedup 1.0000x reference)
import functools

import jax
import jax.numpy as jnp
from jax import lax
from jax.experimental import pallas as pl
from jax.experimental.pallas import tpu as pltpu
from jax.experimental.pallas import tpu_sc as plsc

VS = 1000000
ED = 32
NCTX = 5
B = 16384

NC = 2
NS = 16
NW = NC * NS
BPW = B // NW
CPW = BPW * NCTX
LANES = 16
NSLOT = BPW + CPW


def _sc_body(tgt_hbm, ctx_hbm, ttT_hbm, ctT_hbm, out_hbm,
             tidx, cidx, vals, outv, sem):
    wid = lax.axis_index("s") * NC + lax.axis_index("c")
    tbase = wid * BPW
    cbase = wid * CPW

    pltpu.sync_copy(tgt_hbm.at[pl.ds(tbase, BPW)], tidx)
    pltpu.sync_copy(ctx_hbm.at[pl.ds(cbase, CPW)], cidx)

    copies = []
    for e in range(ED):
        copies.append(pltpu.async_copy(
            ttT_hbm.at[e].at[tidx], vals.at[e, pl.ds(0, BPW)], sem))
        copies.append(pltpu.async_copy(
            ctT_hbm.at[e].at[cidx], vals.at[e, pl.ds(BPW, CPW)], sem))
    for c in copies:
        c.wait()

    iota = lax.broadcasted_iota(jnp.int32, (LANES,), 0)

    def tile_body(t, carry):
        rows = t * LANES + iota
        accs = [jnp.zeros((LANES,), jnp.float32) for _ in range(NCTX)]
        pair0 = rows * NCTX
        for e in range(ED):
            e_vec = jnp.full((LANES,), e, jnp.int32)
            we = plsc.load_gather(vals, [e_vec, rows])
            for c in range(NCTX):
                ce = plsc.load_gather(vals, [e_vec, BPW + pair0 + c])
                accs[c] = accs[c] + we * ce
        for c in range(NCTX):
            plsc.store_scatter(outv, [pair0 + c], accs[c])
        return carry

    lax.fori_loop(0, BPW // LANES, tile_body, 0)

    pltpu.sync_copy(outv, out_hbm.at[pl.ds(cbase, CPW)])


@jax.jit
def _sc_call(tgt_flat, ctx_flat, tt_T, ct_T):
    mesh = plsc.VectorSubcoreMesh(core_axis_name="c", subcore_axis_name="s")
    fn = functools.partial(
        pl.kernel, mesh=mesh,
        out_type=jax.ShapeDtypeStruct((B * NCTX,), jnp.float32),
        scratch_types=[
            pltpu.VMEM((BPW,), jnp.int32),
            pltpu.VMEM((CPW,), jnp.int32),
            pltpu.VMEM((ED, NSLOT), jnp.float32),
            pltpu.VMEM((CPW,), jnp.float32),
            pltpu.SemaphoreType.DMA,
        ],
        compiler_params=pltpu.CompilerParams(
            needs_layout_passes=False, use_tc_tiling_on_sc=False),
    )(_sc_body)
    return fn(tgt_flat, ctx_flat, tt_T, ct_T)


def kernel(target, context, target_table, context_table):
    tgt_flat = target.reshape(B)
    ctx_flat = context.reshape(B * NCTX)
    out_flat = _sc_call(tgt_flat, ctx_flat, target_table.T, context_table.T)
    return out_flat.reshape(B, NCTX)


# speedup vs baseline: 14.4195x; 14.4195x over previous
import functools

import jax
import jax.numpy as jnp
from jax import lax
from jax.experimental import pallas as pl
from jax.experimental.layout import Layout, with_layout_constraint
from jax.experimental.pallas import tpu as pltpu
from jax.experimental.pallas import tpu_sc as plsc

VS = 1000000
ED = 32
NCTX = 5
B = 16384

NC = 2
NS = 16
NW = NC * NS
BPW = B // NW
CPW = BPW * NCTX
LANES = 16
NSLOT = BPW + CPW


def _sc_body(tgt_hbm, ctx_hbm, ttT_hbm, ctT_hbm, out_hbm,
             tidx, cidx, vals, outv, sem):
    wid = lax.axis_index("s") * NC + lax.axis_index("c")
    tbase = wid * BPW
    cbase = wid * CPW

    pltpu.sync_copy(tgt_hbm.at[pl.ds(tbase, BPW)], tidx)
    pltpu.sync_copy(ctx_hbm.at[pl.ds(cbase, CPW)], cidx)

    copies = []
    for e in range(ED):
        copies.append(pltpu.async_copy(
            ttT_hbm.at[e].at[tidx], vals.at[e, pl.ds(0, BPW)], sem))
        copies.append(pltpu.async_copy(
            ctT_hbm.at[e].at[cidx], vals.at[e, pl.ds(BPW, CPW)], sem))
    for c in copies:
        c.wait()

    iota = lax.broadcasted_iota(jnp.int32, (LANES,), 0)

    def tile_body(t, carry):
        rows = t * LANES + iota
        accs = [jnp.zeros((LANES,), jnp.float32) for _ in range(NCTX)]
        pair0 = rows * NCTX
        for e in range(ED):
            e_vec = jnp.full((LANES,), e, jnp.int32)
            we = plsc.load_gather(vals, [e_vec, rows])
            for c in range(NCTX):
                ce = plsc.load_gather(vals, [e_vec, BPW + pair0 + c])
                accs[c] = accs[c] + we * ce
        for c in range(NCTX):
            plsc.store_scatter(outv, [pair0 + c], accs[c])
        return carry

    lax.fori_loop(0, BPW // LANES, tile_body, 0)

    pltpu.sync_copy(outv, out_hbm.at[pl.ds(cbase, CPW)])


@jax.jit
def _sc_call(tgt_flat, ctx_flat, tt_T, ct_T):
    mesh = plsc.VectorSubcoreMesh(core_axis_name="c", subcore_axis_name="s")
    fn = functools.partial(
        pl.kernel, mesh=mesh,
        out_type=jax.ShapeDtypeStruct((B * NCTX,), jnp.float32),
        scratch_types=[
            pltpu.VMEM((BPW,), jnp.int32),
            pltpu.VMEM((CPW,), jnp.int32),
            pltpu.VMEM((ED, NSLOT), jnp.float32),
            pltpu.VMEM((CPW,), jnp.float32),
            pltpu.SemaphoreType.DMA,
        ],
        compiler_params=pltpu.CompilerParams(
            needs_layout_passes=False, use_tc_tiling_on_sc=False),
    )(_sc_body)
    return fn(tgt_flat, ctx_flat, tt_T, ct_T)


def kernel(target, context, target_table, context_table):
    tgt_flat = target.reshape(B)
    ctx_flat = context.reshape(B * NCTX)
    sc_fmt = Layout(major_to_minor=(0, 1), tiling=((8,),))
    tt_T = with_layout_constraint(target_table.T, sc_fmt)
    ct_T = with_layout_constraint(context_table.T, sc_fmt)
    out_flat = _sc_call(tgt_flat, ctx_flat, tt_T, ct_T)
    return out_flat.reshape(B, NCTX)
